# baseline (speedup 1.0000x reference)
import functools

import jax
import jax.numpy as jnp
from jax import lax
from jax.experimental import pallas as pl
from jax.experimental.pallas import tpu as pltpu
from jax.experimental.pallas import tpu_sc as plsc

_VOCAB = 1_000_000
_D = 64
_B = 4096 * 200
_NW = 32
_BPW = _B // _NW
_CH = 256
_NCH = _BPW // _CH
_NBUF = 2
_NSTEP = _NCH // _NBUF

_mesh = plsc.VectorSubcoreMesh(core_axis_name="c", subcore_axis_name="s")

_NBLK = 7812
_TBUF = 2


@functools.partial(
    pl.kernel,
    out_type=jax.ShapeDtypeStruct((_VOCAB // 2, 128), jnp.float32),
    mesh=_mesh,
    scratch_types=[
        [pltpu.VMEM((_D, 128), jnp.float32) for _ in range(_TBUF)],
        [pltpu.VMEM((_D, 128), jnp.float32) for _ in range(_TBUF)],
        [pltpu.SemaphoreType.DMA for _ in range(_TBUF)],
        [pltpu.SemaphoreType.DMA for _ in range(_TBUF)],
    ],
    compiler_params=pltpu.CompilerParams(
        use_tc_tiling_on_sc=True, needs_layout_passes=False),
)
def _transpose_kernel(tableT_hbm, tail_hbm, out_hbm, src_v, dst_v, sem_l, sem_s):
    wid = lax.axis_index("s") * 2 + lax.axis_index("c")
    nfull = _NBLK // _NW
    nrem = _NBLK - nfull * _NW

    def load_copy(b, t):
        return pltpu.make_async_copy(
            tableT_hbm.at[:, pl.ds(t * 128, 128)], src_v[b], sem_l[b])

    def store_copy(b, t):
        return pltpu.make_async_copy(
            dst_v[b], out_hbm.at[pl.ds(t * 64, _D)], sem_s[b])

    iota16 = lax.iota(jnp.int32, 16)
    rows_static = [iota16 + 16 * (v % 4) for v in range(8)]

    def transpose_block(b):
        for p2 in range(_D):
            for v in range(8):
                cols = jnp.full((16,), 2 * p2 + (v // 4), jnp.int32)
                vals = plsc.load_gather(src_v[b], [rows_static[v], cols])
                dst_v[b][p2, pl.ds(16 * v, 16)] = vals

    def blk(k):
        return k * _NW + wid

    for b in range(_TBUF):
        load_copy(b, blk(b)).start()

    def body(g, carry):
        for b in range(_TBUF):
            k_cur = g * _TBUF + b
            k_nxt = (g + 1) * _TBUF + b
            load_copy(b, blk(k_cur)).wait()

            @pl.when(g > 0)
            def _():
                store_copy(b, blk(k_cur - _TBUF)).wait()

            transpose_block(b)
            store_copy(b, blk(k_cur)).start()

            @pl.when(k_nxt < nfull)
            def _():
                load_copy(b, blk(k_nxt)).start()
        return carry

    lax.fori_loop(0, nfull // _TBUF, body, 0)
    for b in range(_TBUF):
        store_copy(b, blk(nfull - _TBUF + b)).wait()

    @pl.when(wid < nrem)
    def _():
        t = nfull * _NW + wid
        pltpu.sync_copy(tableT_hbm.at[:, pl.ds(t * 128, 128)], src_v[0])
        transpose_block(0)
        pltpu.sync_copy(dst_v[0], out_hbm.at[pl.ds(t * 64, _D)])

    @pl.when(wid == _NW - 1)
    def _():
        pltpu.sync_copy(tail_hbm, src_v[0].at[pl.ds(0, 32), :])
        pltpu.sync_copy(src_v[0].at[pl.ds(0, 32), :],
                        out_hbm.at[pl.ds(_NBLK * 64, 32)])


@functools.partial(
    pl.kernel,
    out_type=jax.ShapeDtypeStruct((_B, 128), jnp.float32),
    mesh=_mesh,
    scratch_types=[
        [pltpu.VMEM((_CH,), jnp.int32) for _ in range(_NBUF)],
        [pltpu.VMEM((_CH,), jnp.int32) for _ in range(_NBUF)],
        [pltpu.VMEM((_CH, 128), jnp.float32) for _ in range(_NBUF)],
        [pltpu.SemaphoreType.DMA for _ in range(_NBUF)],
        [pltpu.SemaphoreType.DMA for _ in range(_NBUF)],
        [pltpu.SemaphoreType.DMA for _ in range(_NBUF)],
        [pltpu.SemaphoreType.DMA for _ in range(_NBUF)],
    ],
    compiler_params=pltpu.CompilerParams(
        use_tc_tiling_on_sc=True, needs_layout_passes=False),
)
def _gather_kernel(idx_hbm, off_hbm, table_hbm, out_hbm,
                   idx_v, off_v, rows_v, sem_i, sem_o, sem_g, sem_s):
    wid = lax.axis_index("s") * 2 + lax.axis_index("c")
    base = wid * _BPW

    def idx_copy(b, off):
        return pltpu.make_async_copy(
            idx_hbm.at[pl.ds(off, _CH)], idx_v[b], sem_i[b])

    def off_copy(b, off):
        return pltpu.make_async_copy(
            off_hbm.at[pl.ds(off, _CH)], off_v[b], sem_o[b])

    def gather_copy(b):
        return pltpu.make_async_copy(table_hbm.at[idx_v[b]], rows_v[b], sem_g[b])

    def store_copy(b, off):
        return pltpu.make_async_copy(
            rows_v[b], out_hbm.at[pl.ds(off, _CH)], sem_s[b])

    def half_select(b):
        iota16 = lax.iota(jnp.int32, 16)

        def grp_body(g2, carry):
            ovec = off_v[b][pl.ds(g2 * 16, 16)]
            for lane in range(16):
                r = g2 * 16 + lane
                o = ovec[lane]
                for d0 in range(0, _D, 16):
                    lanes = o + d0 + iota16
                    vals = plsc.load_gather(
                        rows_v[b], [jnp.full((16,), r, jnp.int32), lanes])
                    rows_v[b][r, pl.ds(d0, 16)] = vals
            return carry

        lax.fori_loop(0, _CH // 16, grp_body, 0)

    for b in range(_NBUF):
        idx_copy(b, base + b * _CH).start()
        off_copy(b, base + b * _CH).start()
    for b in range(_NBUF):
        idx_copy(b, base + b * _CH).wait()
        gather_copy(b).start()

    def body(g, carry):
        for b in range(_NBUF):
            off_prev = base + ((g - 1) * _NBUF + b) * _CH
            off_new = base + (g * _NBUF + b) * _CH
            gather_copy(b).wait()
            off_copy(b, off_prev).wait()
            half_select(b)
            store_copy(b, off_prev).start()
            idx_copy(b, off_new).start()
            store_copy(b, off_prev).wait()
            idx_copy(b, off_new).wait()
            off_copy(b, off_new).start()
            gather_copy(b).start()
        return carry

    lax.fori_loop(1, _NSTEP, body, 0)

    for b in range(_NBUF):
        off = base + ((_NSTEP - 1) * _NBUF + b) * _CH
        gather_copy(b).wait()
        off_copy(b, off).wait()
        half_select(b)
        store_copy(b, off).start()
    for b in range(_NBUF):
        off = base + ((_NSTEP - 1) * _NBUF + b) * _CH
        store_copy(b, off).wait()


def kernel(x, embedding_table, training, mask):
    xi = x.reshape(-1).astype(jnp.int32)
    idxp = xi >> 1
    off = (xi & 1) * _D
    tableT = jnp.swapaxes(embedding_table, 0, 1)
    tail = embedding_table[_NBLK * 128:].reshape(32, 128)
    t128 = _transpose_kernel(tableT, tail)
    out = _gather_kernel(idxp, off, t128)
    return out[:, :_D].reshape(x.shape[0], x.shape[1], _D)


# speedup vs baseline: 1.9340x; 1.9340x over previous
import functools

import jax
import jax.numpy as jnp
from jax import lax
from jax.experimental import pallas as pl
from jax.experimental.pallas import tpu as pltpu
from jax.experimental.pallas import tpu_sc as plsc

_VOCAB = 1_000_000
_D = 64
_B = 4096 * 200
_NW = 32
_BPW = _B // _NW
_CH = 512
_NCH = _BPW // _CH
_NBUF = 2
_NSTEP = _NCH // _NBUF

_mesh = plsc.VectorSubcoreMesh(core_axis_name="c", subcore_axis_name="s")


@functools.partial(
    pl.kernel,
    out_type=jax.ShapeDtypeStruct((_B, _D), jnp.float32),
    mesh=_mesh,
    scratch_types=[
        [pltpu.VMEM((_CH,), jnp.int32) for _ in range(_NBUF)],
        [pltpu.VMEM((_CH, _D), jnp.float32) for _ in range(_NBUF)],
        [pltpu.SemaphoreType.DMA for _ in range(_NBUF)],
        [pltpu.SemaphoreType.DMA for _ in range(_NBUF)],
        [pltpu.SemaphoreType.DMA for _ in range(_NBUF)],
    ],
    compiler_params=pltpu.CompilerParams(use_tc_tiling_on_sc=False),
)
def _gather_kernel(idx_hbm, table_hbm, out_hbm, idx_v, rows_v, sem_i, sem_g, sem_s):
    wid = lax.axis_index("s") * 2 + lax.axis_index("c")
    base = wid * _BPW

    def idx_copy(b, off):
        return pltpu.make_async_copy(
            idx_hbm.at[pl.ds(off, _CH)], idx_v[b], sem_i[b])

    def gather_copy(b):
        return pltpu.make_async_copy(table_hbm.at[idx_v[b]], rows_v[b], sem_g[b])

    def store_copy(b, off):
        return pltpu.make_async_copy(
            rows_v[b], out_hbm.at[pl.ds(off, _CH)], sem_s[b])

    for b in range(_NBUF):
        idx_copy(b, base + b * _CH).start()
    for b in range(_NBUF):
        idx_copy(b, base + b * _CH).wait()
        gather_copy(b).start()

    def body(g, carry):
        for b in range(_NBUF):
            off_prev = base + ((g - 1) * _NBUF + b) * _CH
            off_new = base + (g * _NBUF + b) * _CH
            gather_copy(b).wait()
            store_copy(b, off_prev).start()
            idx_copy(b, off_new).start()
            store_copy(b, off_prev).wait()
            idx_copy(b, off_new).wait()
            gather_copy(b).start()
        return carry

    lax.fori_loop(1, _NSTEP, body, 0)

    for b in range(_NBUF):
        off = base + ((_NSTEP - 1) * _NBUF + b) * _CH
        gather_copy(b).wait()
        store_copy(b, off).start()
    for b in range(_NBUF):
        off = base + ((_NSTEP - 1) * _NBUF + b) * _CH
        store_copy(b, off).wait()


def kernel(x, embedding_table, training, mask):
    idx = x.reshape(-1).astype(jnp.int32)
    out = _gather_kernel(idx, embedding_table)
    return out.reshape(x.shape[0], x.shape[1], _D)
